# Initial kernel scaffold; baseline (speedup 1.0000x reference)
import jax
import jax.numpy as jnp
from jax.experimental import pallas as pl


def kernel(x, edge_index, batch, W1, b1, W2, b2, fc_w, fc_b):
    raise NotImplementedError("write your pallas kernel here")


# speedup vs baseline: 8.7882x; 8.7882x over previous
import functools

import jax
import jax.numpy as jnp
from jax import lax
from jax.experimental import pallas as pl
from jax.experimental.pallas import tpu as pltpu
from jax.experimental.pallas import tpu_sc as plsc

N = 10000
E = 320000
G = 64
NPAD = 10240
NC, NS = 2, 16
NW = NC * NS
EPW = E // NW
CH = 80
NCHUNK = EPW // CH
RPT = NPAD // NS
FD = 16

_mesh = plsc.VectorSubcoreMesh(core_axis_name="c", subcore_axis_name="s")
_sc_params = pltpu.CompilerParams(use_tc_tiling_on_sc=False)


def _make_lap(F):
    @functools.partial(
        pl.kernel,
        out_type=jax.ShapeDtypeStruct((NC, NPAD, F), jnp.float32),
        mesh=_mesh,
        scratch_types=[
            pltpu.VMEM_SHARED((NPAD, F), jnp.float32),
            pltpu.VMEM((CH,), jnp.int32),
            pltpu.VMEM((CH,), jnp.int32),
            pltpu.VMEM((CH, F), jnp.float32),
            pltpu.SemaphoreType.DMA,
        ],
        compiler_params=_sc_params,
    )
    def lap(src_hbm, dst_hbm, table_hbm, zeros_hbm, out_hbm,
            acc, src_v, dst_v, rows_v, sem):
        c = lax.axis_index("c")
        s = lax.axis_index("s")
        w = s * NC + c
        row0 = s * RPT
        pltpu.sync_copy(zeros_hbm.at[pl.ds(row0, RPT)], acc.at[pl.ds(row0, RPT)])
        plsc.subcore_barrier()
        base_e = w * EPW

        def body(i, carry):
            e0 = base_e + i * CH
            pltpu.sync_copy(src_hbm.at[pl.ds(e0, CH)], src_v)
            pltpu.sync_copy(dst_hbm.at[pl.ds(e0, CH)], dst_v)
            pltpu.async_copy(table_hbm.at[src_v], rows_v, sem).wait()
            pltpu.sync_copy(rows_v, acc.at[dst_v], add=True)
            return carry

        lax.fori_loop(0, NCHUNK, body, 0)
        plsc.subcore_barrier()
        pltpu.sync_copy(acc.at[pl.ds(row0, RPT)], out_hbm.at[c, pl.ds(row0, RPT)])

    return lap


_lap64 = _make_lap(64)
_lap32 = _make_lap(32)


@functools.partial(
    pl.kernel,
    out_type=jax.ShapeDtypeStruct((NC, NPAD, FD), jnp.float32),
    mesh=_mesh,
    scratch_types=[
        pltpu.VMEM_SHARED((NPAD, FD), jnp.float32),
        pltpu.VMEM((CH,), jnp.int32),
        pltpu.VMEM((CH, FD), jnp.float32),
    ],
    compiler_params=_sc_params,
)
def _deg_sc(dst_hbm, zeros_hbm, ones_hbm, out_hbm, acc, dst_v, ones_v):
    c = lax.axis_index("c")
    s = lax.axis_index("s")
    w = s * NC + c
    row0 = s * RPT
    pltpu.sync_copy(zeros_hbm.at[pl.ds(row0, RPT)], acc.at[pl.ds(row0, RPT)])
    pltpu.sync_copy(ones_hbm, ones_v)
    plsc.subcore_barrier()
    base_e = w * EPW

    def body(i, carry):
        e0 = base_e + i * CH
        pltpu.sync_copy(dst_hbm.at[pl.ds(e0, CH)], dst_v)
        pltpu.sync_copy(ones_v, acc.at[dst_v], add=True)
        return carry

    lax.fori_loop(0, NCHUNK, body, 0)
    plsc.subcore_barrier()
    pltpu.sync_copy(acc.at[pl.ds(row0, RPT)], out_hbm.at[c, pl.ds(row0, RPT)])


def _dot(a, b):
    return jnp.dot(a, b, preferred_element_type=jnp.float32)


def _tc1_body(p_ref, x_ref, w_ref, dis_ref, a_ref, u0_ref, u2_ref):
    deg = p_ref[0][:N, 0:1] + p_ref[1][:N, 0:1]
    dis = jnp.where(deg > 0, lax.rsqrt(jnp.maximum(deg, 1e-12)), 0.0)
    u = _dot(x_ref[...], w_ref[...])
    u1 = u[:, 32:64]
    u2 = u[:, 64:96]
    dis_ref[...] = dis
    a_ref[...] = jnp.concatenate([dis * u1, dis * u2], axis=1)
    u0_ref[...] = u[:, 0:32]
    u2_ref[...] = u2


def _tc2_body(p_ref, dis_ref, s1_ref, t3_ref):
    sp = p_ref[0][:N] + p_ref[1][:N]
    dis = dis_ref[...]
    s1_ref[...] = sp[:, 0:32]
    t3_ref[...] = (dis * dis) * sp[:, 32:64]


def _tc3_body(u0_ref, u2_ref, s1_ref, q_ref, dis_ref, b_ref,
              w0_ref, w1_ref, w2_ref, a_ref, v0_ref, v2_ref):
    dis = dis_ref[...]
    s3 = q_ref[0][:N] + q_ref[1][:N]
    h = jax.nn.relu(u0_ref[...] - dis * s1_ref[...] + 2.0 * dis * s3
                    - u2_ref[...] + b_ref[...])
    v1 = _dot(h, w1_ref[...])
    v2 = _dot(h, w2_ref[...])
    a_ref[...] = jnp.concatenate([dis * v1, dis * v2], axis=1)
    v0_ref[...] = _dot(h, w0_ref[...])
    v2_ref[...] = v2


def _tc5_body(v0_ref, v2_ref, s4_ref, q_ref, dis_ref, b_ref, fcw_ref,
              fcb_ref, batch_ref, out_ref):
    dis = dis_ref[...]
    s6 = q_ref[0][:N] + q_ref[1][:N]
    h = jax.nn.relu(v0_ref[...] - dis * s4_ref[...] + 2.0 * dis * s6
                    - v2_ref[...] + b_ref[...])
    r = _dot(h, fcw_ref[...])
    gid = lax.broadcasted_iota(jnp.int32, (G, N), 0)
    m = (batch_ref[...] == gid).astype(jnp.float32)
    pooled = _dot(m, r)
    cnt = jnp.sum(m, axis=1, keepdims=True)
    out_ref[...] = pooled / jnp.maximum(cnt, 1.0) + fcb_ref[...]


def _f32(shape):
    return jax.ShapeDtypeStruct(shape, jnp.float32)


_tc1 = pl.pallas_call(
    _tc1_body, out_shape=(_f32((N, 1)), _f32((N, 64)), _f32((N, 32)), _f32((N, 32))))
_tc2 = pl.pallas_call(_tc2_body, out_shape=(_f32((N, 32)), _f32((N, 32))))
_tc3 = pl.pallas_call(
    _tc3_body, out_shape=(_f32((N, 64)), _f32((N, 32)), _f32((N, 32))))
_tc5 = pl.pallas_call(_tc5_body, out_shape=_f32((G, 1)))


def kernel(x, edge_index, batch, W1, b1, W2, b2, fc_w, fc_b):
    src = edge_index[0]
    dst = edge_index[1]
    w1all = jnp.concatenate([W1[0], W1[1], W1[2]], axis=1)
    z64 = jnp.zeros((NPAD, 64), jnp.float32)
    z32 = jnp.zeros((NPAD, 32), jnp.float32)
    z16 = jnp.zeros((NPAD, FD), jnp.float32)
    ones16 = jnp.ones((CH, FD), jnp.float32)

    degp = _deg_sc(dst, z16, ones16)
    dis, a, u0, u2 = _tc1(degp, x, w1all)
    p1 = _lap64(src, dst, a, z64)
    s1, t3 = _tc2(p1, dis)
    q1 = _lap32(src, dst, t3, z32)
    bt, v0, v2 = _tc3(u0, u2, s1, q1, dis, b1.reshape(1, 32),
                      W2[0], W2[1], W2[2])
    p2 = _lap64(src, dst, bt, z64)
    s4, t6 = _tc2(p2, dis)
    q2 = _lap32(src, dst, t6, z32)
    out = _tc5(v0, v2, s4, q2, dis, b2.reshape(1, 32), fc_w,
               fc_b.reshape(1, 1), batch.reshape(1, N))
    return out.reshape(G)


# speedup vs baseline: 9.8089x; 1.1161x over previous
import functools

import jax
import jax.numpy as jnp
from jax import lax
from jax.experimental import pallas as pl
from jax.experimental.pallas import tpu as pltpu
from jax.experimental.pallas import tpu_sc as plsc

N = 10000
E = 320000
G = 64
NPAD = 10240
NC, NS = 2, 16
NW = NC * NS
CH = 128
EPAD = 10240
NCH = EPAD // CH
KB = 8
NG = NCH // KB
RPT = NPAD // NS
FD = 16

_mesh = plsc.VectorSubcoreMesh(core_axis_name="c", subcore_axis_name="s")
_sc_params = pltpu.CompilerParams(use_tc_tiling_on_sc=False)


def _make_lap(F):
    @functools.partial(
        pl.kernel,
        out_type=jax.ShapeDtypeStruct((NC, NPAD, F), jnp.float32),
        mesh=_mesh,
        scratch_types=[
            pltpu.VMEM_SHARED((NPAD, F), jnp.float32),
            pltpu.VMEM((NCH, CH), jnp.int32),
            pltpu.VMEM((NCH, CH), jnp.int32),
            pltpu.VMEM((KB, CH, F), jnp.float32),
            pltpu.SemaphoreType.DMA,
            pltpu.SemaphoreType.DMA,
        ],
        compiler_params=_sc_params,
    )
    def lap(src_hbm, dst_hbm, table_hbm, zeros_hbm, out_hbm,
            acc, src_v, dst_v, rows, gsem, ssem):
        c = lax.axis_index("c")
        s = lax.axis_index("s")
        w = s * NC + c
        row0 = s * RPT
        pltpu.sync_copy(zeros_hbm.at[pl.ds(row0, RPT)], acc.at[pl.ds(row0, RPT)])
        pltpu.sync_copy(src_hbm.at[w], src_v)
        pltpu.sync_copy(dst_hbm.at[w], dst_v)
        plsc.subcore_barrier()

        def group(g, carry):
            j0 = g * KB
            for b in range(KB):
                pltpu.async_copy(table_hbm.at[src_v.at[j0 + b]], rows.at[b], gsem)
            for b in range(KB):
                pltpu.make_async_copy(
                    table_hbm.at[src_v.at[j0 + b]], rows.at[b], gsem).wait()
            for b in range(KB):
                pltpu.async_copy(rows.at[b], acc.at[dst_v.at[j0 + b]], ssem,
                                 add=True)
            for b in range(KB):
                pltpu.make_async_copy(
                    rows.at[b], acc.at[dst_v.at[j0 + b]], ssem).wait()
            return carry

        lax.fori_loop(0, NG, group, 0)
        plsc.subcore_barrier()
        pltpu.sync_copy(acc.at[pl.ds(row0, RPT)], out_hbm.at[c, pl.ds(row0, RPT)])

    return lap


_lap64 = _make_lap(64)
_lap32 = _make_lap(32)


@functools.partial(
    pl.kernel,
    out_type=jax.ShapeDtypeStruct((NC, NPAD, FD), jnp.float32),
    mesh=_mesh,
    scratch_types=[
        pltpu.VMEM_SHARED((NPAD, FD), jnp.float32),
        pltpu.VMEM((NCH, CH), jnp.int32),
        pltpu.VMEM((CH, FD), jnp.float32),
        pltpu.SemaphoreType.DMA,
    ],
    compiler_params=_sc_params,
)
def _deg_sc(dst_hbm, zeros_hbm, ones_hbm, out_hbm, acc, dst_v, ones_v, ssem):
    c = lax.axis_index("c")
    s = lax.axis_index("s")
    w = s * NC + c
    row0 = s * RPT
    pltpu.sync_copy(zeros_hbm.at[pl.ds(row0, RPT)], acc.at[pl.ds(row0, RPT)])
    pltpu.sync_copy(ones_hbm, ones_v)
    pltpu.sync_copy(dst_hbm.at[w], dst_v)
    plsc.subcore_barrier()

    def group(g, carry):
        j0 = g * KB
        for b in range(KB):
            pltpu.async_copy(ones_v, acc.at[dst_v.at[j0 + b]], ssem, add=True)
        for b in range(KB):
            pltpu.make_async_copy(ones_v, acc.at[dst_v.at[j0 + b]], ssem).wait()
        return carry

    lax.fori_loop(0, NG, group, 0)
    plsc.subcore_barrier()
    pltpu.sync_copy(acc.at[pl.ds(row0, RPT)], out_hbm.at[c, pl.ds(row0, RPT)])


def _dot(a, b):
    return jnp.dot(a, b, preferred_element_type=jnp.float32)


def _tc1_body(p_ref, x_ref, w_ref, dis_ref, a_ref, u0_ref, u2_ref):
    deg = p_ref[0][:N, 0:1] + p_ref[1][:N, 0:1]
    dis = jnp.where(deg > 0, lax.rsqrt(jnp.maximum(deg, 1e-12)), 0.0)
    u = _dot(x_ref[...], w_ref[...])
    u1 = u[:, 32:64]
    u2 = u[:, 64:96]
    dis_ref[...] = dis
    a_ref[...] = jnp.concatenate([dis * u1, dis * u2], axis=1)
    u0_ref[...] = u[:, 0:32]
    u2_ref[...] = u2


def _tc2_body(p_ref, dis_ref, s1_ref, t3_ref):
    sp = p_ref[0][:N] + p_ref[1][:N]
    dis = dis_ref[...]
    s1_ref[...] = sp[:, 0:32]
    t3_ref[...] = (dis * dis) * sp[:, 32:64]


def _tc3_body(u0_ref, u2_ref, s1_ref, q_ref, dis_ref, b_ref,
              w0_ref, w1_ref, w2_ref, a_ref, v0_ref, v2_ref):
    dis = dis_ref[...]
    s3 = q_ref[0][:N] + q_ref[1][:N]
    h = jax.nn.relu(u0_ref[...] - dis * s1_ref[...] + 2.0 * dis * s3
                    - u2_ref[...] + b_ref[...])
    v1 = _dot(h, w1_ref[...])
    v2 = _dot(h, w2_ref[...])
    a_ref[...] = jnp.concatenate([dis * v1, dis * v2], axis=1)
    v0_ref[...] = _dot(h, w0_ref[...])
    v2_ref[...] = v2


def _tc5_body(v0_ref, v2_ref, s4_ref, q_ref, dis_ref, b_ref, fcw_ref,
              fcb_ref, batch_ref, out_ref):
    dis = dis_ref[...]
    s6 = q_ref[0][:N] + q_ref[1][:N]
    h = jax.nn.relu(v0_ref[...] - dis * s4_ref[...] + 2.0 * dis * s6
                    - v2_ref[...] + b_ref[...])
    r = _dot(h, fcw_ref[...])
    gid = lax.broadcasted_iota(jnp.int32, (G, N), 0)
    m = (batch_ref[...] == gid).astype(jnp.float32)
    pooled = _dot(m, r)
    cnt = jnp.sum(m, axis=1, keepdims=True)
    out_ref[...] = pooled / jnp.maximum(cnt, 1.0) + fcb_ref[...]


def _f32(shape):
    return jax.ShapeDtypeStruct(shape, jnp.float32)


_tc1 = pl.pallas_call(
    _tc1_body, out_shape=(_f32((N, 1)), _f32((N, 64)), _f32((N, 32)), _f32((N, 32))))
_tc2 = pl.pallas_call(_tc2_body, out_shape=(_f32((N, 32)), _f32((N, 32))))
_tc3 = pl.pallas_call(
    _tc3_body, out_shape=(_f32((N, 64)), _f32((N, 32)), _f32((N, 32))))
_tc5 = pl.pallas_call(_tc5_body, out_shape=_f32((G, 1)))


def kernel(x, edge_index, batch, W1, b1, W2, b2, fc_w, fc_b):
    npad_e = NW * EPAD - E
    src = jnp.concatenate(
        [edge_index[0], jnp.zeros((npad_e,), jnp.int32)]).reshape(NW, NCH, CH)
    dst = jnp.concatenate(
        [edge_index[1], jnp.full((npad_e,), NPAD - 1, jnp.int32)]
    ).reshape(NW, NCH, CH)
    w1all = jnp.concatenate([W1[0], W1[1], W1[2]], axis=1)
    z64 = jnp.zeros((NPAD, 64), jnp.float32)
    z32 = jnp.zeros((NPAD, 32), jnp.float32)
    z16 = jnp.zeros((NPAD, FD), jnp.float32)
    ones16 = jnp.ones((CH, FD), jnp.float32)

    degp = _deg_sc(dst, z16, ones16)
    dis, a, u0, u2 = _tc1(degp, x, w1all)
    p1 = _lap64(src, dst, a, z64)
    s1, t3 = _tc2(p1, dis)
    q1 = _lap32(src, dst, t3, z32)
    bt, v0, v2 = _tc3(u0, u2, s1, q1, dis, b1.reshape(1, 32),
                      W2[0], W2[1], W2[2])
    p2 = _lap64(src, dst, bt, z64)
    s4, t6 = _tc2(p2, dis)
    q2 = _lap32(src, dst, t6, z32)
    out = _tc5(v0, v2, s4, q2, dis, b2.reshape(1, 32), fc_w,
               fc_b.reshape(1, 1), batch.reshape(1, N))
    return out.reshape(G)
